# baseline (speedup 1.0000x reference)
import jax
import jax.numpy as jnp
from jax import lax
from jax.experimental import pallas as pl
from jax.experimental.pallas import tpu as pltpu
from jax.experimental.pallas import tpu_sc as plsc

_B = 4
_S = 4096
_D = 4096
_NW = 32
_S_PER_W = _S // _NW
_CHUNK = 8
_N_CHUNKS = _S_PER_W // _CHUNK
_SCALE = 64.0
_LANES = 16


def _body(ids_hbm, table_hbm, pos_hbm, out_hbm,
          idx_v, pos_v, rows0, rows1,
          gsem0, gsem1, ssem0, ssem1, psem):
    wid = lax.axis_index("s") * 2 + lax.axis_index("c")
    s0 = wid * _S_PER_W

    rows = (rows0, rows1)
    gsem = (gsem0, gsem1)
    ssem = (ssem0, ssem1)

    for b in range(_B):
        pltpu.sync_copy(ids_hbm.at[b, pl.ds(s0, _S_PER_W)], idx_v.at[b])

    def gather(c, b, buf_i):
        return pltpu.make_async_copy(
            table_hbm.at[idx_v.at[b, pl.ds(c * _CHUNK, _CHUNK)]],
            rows[buf_i],
            gsem[buf_i],
        )

    def store(c, b, buf_i):
        return pltpu.make_async_copy(
            rows[buf_i],
            out_hbm.at[b, pl.ds(s0 + c * _CHUNK, _CHUNK), :],
            ssem[buf_i],
        )

    def pos_load(c):
        return pltpu.make_async_copy(
            pos_hbm.at[pl.ds(s0 + c * _CHUNK, _CHUNK), :],
            pos_v,
            psem,
        )

    pos_load(0).start()
    gather(0, 0, 0).start()

    def chunk_body(c, carry):
        for b in range(_B):
            buf_i = b % 2
            other = 1 - buf_i
            gather(c, b, buf_i).wait()
            if b == 0:
                @pl.when(c > 0)
                def _():
                    store(c, b, other).wait()
                pos_load(c).wait()
            else:
                store(c, b, other).wait()
            if b < _B - 1:
                gather(c, b + 1, other).start()
            else:
                cn = jnp.minimum(c + 1, _N_CHUNKS - 1)
                gather(cn, 0, other).start()

            buf = rows[buf_i]

            def fma(j, acc):
                off = j * _LANES
                for r in range(_CHUNK):
                    buf[r, pl.ds(off, _LANES)] = (
                        buf[r, pl.ds(off, _LANES)] * _SCALE
                        + pos_v[r, pl.ds(off, _LANES)]
                    )
                return acc

            lax.fori_loop(0, _D // _LANES, fma, 0, unroll=4)

            store(c, b, buf_i).start()
            if b == _B - 1:
                cn = jnp.minimum(c + 1, _N_CHUNKS - 1)
                pos_load(cn).start()
        return carry

    lax.fori_loop(0, _N_CHUNKS, chunk_body, 0)

    gather(_N_CHUNKS - 1, 0, 0).wait()
    store(_N_CHUNKS - 1, _B - 1, 1).wait()
    pos_load(_N_CHUNKS - 1).wait()


@jax.jit
def _embed(token_ids, token_table, pos_table):
    mesh = plsc.VectorSubcoreMesh(core_axis_name="c", subcore_axis_name="s")
    return pl.kernel(
        _body,
        out_type=jax.ShapeDtypeStruct((_B, _S, _D), jnp.float32),
        mesh=mesh,
        scratch_types=[
            pltpu.VMEM((_B, _S_PER_W), jnp.int32),
            pltpu.VMEM((_CHUNK, _D), jnp.float32),
            pltpu.VMEM((_CHUNK, _D), jnp.float32),
            pltpu.VMEM((_CHUNK, _D), jnp.float32),
            pltpu.SemaphoreType.DMA,
            pltpu.SemaphoreType.DMA,
            pltpu.SemaphoreType.DMA,
            pltpu.SemaphoreType.DMA,
            pltpu.SemaphoreType.DMA,
        ],
    )(token_ids, token_table, pos_table)


def kernel(token_ids, token_table, pos_table):
    return _embed(token_ids.astype(jnp.int32), token_table, pos_table)


# speedup vs baseline: 1.5501x; 1.5501x over previous
import jax
import jax.numpy as jnp
from jax import lax
from jax.experimental import pallas as pl
from jax.experimental.pallas import tpu as pltpu
from jax.experimental.pallas import tpu_sc as plsc

_B = 4
_S = 4096
_D = 4096
_NW = 32
_S_PER_W = _S // _NW
_CHUNK = 8
_N_CHUNKS = _S_PER_W // _CHUNK
_SCALE = 64.0
_LANES = 16


def _body(ids_hbm, table_hbm, pos_hbm, out_hbm,
          idx_v, pos_v, rows0, rows1,
          gsem0, gsem1, ssem0, ssem1, psem):
    wid = lax.axis_index("s") * 2 + lax.axis_index("c")
    s0 = wid * _S_PER_W

    rows = (rows0, rows1)
    gsem = (gsem0, gsem1)
    ssem = (ssem0, ssem1)

    for b in range(_B):
        pltpu.sync_copy(ids_hbm.at[b, pl.ds(s0, _S_PER_W)], idx_v.at[b])

    def gather(c, b, buf_i):
        return pltpu.make_async_copy(
            table_hbm.at[idx_v.at[b, pl.ds(c * _CHUNK, _CHUNK)]],
            rows[buf_i],
            gsem[buf_i],
        )

    def store(c, b, buf_i):
        return pltpu.make_async_copy(
            rows[buf_i],
            out_hbm.at[b, pl.ds(s0 + c * _CHUNK, _CHUNK), :],
            ssem[buf_i],
        )

    def pos_load(c):
        return pltpu.make_async_copy(
            pos_hbm.at[pl.ds(s0 + c * _CHUNK, _CHUNK), :],
            pos_v,
            psem,
        )

    pos_load(0).start()
    gather(0, 0, 0).start()

    def chunk_body(c, carry):
        for b in range(_B):
            buf_i = b % 2
            other = 1 - buf_i
            gather(c, b, buf_i).wait()
            if b == 0:
                @pl.when(c > 0)
                def _():
                    store(c, b, other).wait()
                pos_load(c).wait()
            else:
                store(c, b, other).wait()
            if b < _B - 1:
                gather(c, b + 1, other).start()
            else:
                cn = jnp.minimum(c + 1, _N_CHUNKS - 1)
                gather(cn, 0, other).start()

            buf = rows[buf_i]

            def fma(j, acc):
                off = j * _LANES
                for r in range(_CHUNK):
                    buf[r, pl.ds(off, _LANES)] = (
                        buf[r, pl.ds(off, _LANES)] * _SCALE
                        + pos_v[r, pl.ds(off, _LANES)]
                    )
                return acc

            lax.fori_loop(0, _D // _LANES, fma, 0)

            store(c, b, buf_i).start()
            if b == _B - 1:
                cn = jnp.minimum(c + 1, _N_CHUNKS - 1)
                pos_load(cn).start()
        return carry

    lax.fori_loop(0, _N_CHUNKS, chunk_body, 0)

    gather(_N_CHUNKS - 1, 0, 0).wait()
    store(_N_CHUNKS - 1, _B - 1, 1).wait()
    pos_load(_N_CHUNKS - 1).wait()


@jax.jit
def _embed(token_ids, token_table, pos_table):
    mesh = plsc.VectorSubcoreMesh(core_axis_name="c", subcore_axis_name="s")
    return pl.kernel(
        _body,
        out_type=jax.ShapeDtypeStruct((_B, _S, _D), jnp.float32),
        mesh=mesh,
        scratch_types=[
            pltpu.VMEM((_B, _S_PER_W), jnp.int32),
            pltpu.VMEM((_CHUNK, _D), jnp.float32),
            pltpu.VMEM((_CHUNK, _D), jnp.float32),
            pltpu.VMEM((_CHUNK, _D), jnp.float32),
            pltpu.SemaphoreType.DMA,
            pltpu.SemaphoreType.DMA,
            pltpu.SemaphoreType.DMA,
            pltpu.SemaphoreType.DMA,
            pltpu.SemaphoreType.DMA,
        ],
    )(token_ids, token_table, pos_table)


def kernel(token_ids, token_table, pos_table):
    return _embed(token_ids.astype(jnp.int32), token_table, pos_table)


# speedup vs baseline: 2.6329x; 1.6986x over previous
import jax
import jax.numpy as jnp
from jax import lax
from jax.experimental import pallas as pl
from jax.experimental.pallas import tpu as pltpu
from jax.experimental.pallas import tpu_sc as plsc

_B = 4
_S = 4096
_D = 4096
_NW = 32
_S_PER_W = _S // _NW
_CHUNK = 4
_N_CHUNKS = _S_PER_W // _CHUNK
_SCALE = 64.0
_LANES = 16


def _body(ids_hbm, table_hbm, pos_hbm, out_hbm,
          idx_v, rows0, rows1, outb0, outb1, pos0, pos1,
          gsem0, gsem1, ssem0, ssem1, psem0, psem1):
    wid = lax.axis_index("s") * 2 + lax.axis_index("c")
    s0 = wid * _S_PER_W

    rows = (rows0, rows1)
    outb = (outb0, outb1)
    posb = (pos0, pos1)
    gsem = (gsem0, gsem1)
    ssem = (ssem0, ssem1)
    psem = (psem0, psem1)

    for b in range(_B):
        pltpu.sync_copy(ids_hbm.at[b, pl.ds(s0, _S_PER_W)], idx_v.at[b])

    def gather(c, b, p):
        return pltpu.make_async_copy(
            table_hbm.at[idx_v.at[b, pl.ds(c * _CHUNK, _CHUNK)]],
            rows[p],
            gsem[p],
        )

    def store(c, b, p):
        return pltpu.make_async_copy(
            outb[p],
            out_hbm.at[b, pl.ds(s0 + c * _CHUNK, _CHUNK), :],
            ssem[p],
        )

    def pos_load(c, dc):
        return pltpu.make_async_copy(
            pos_hbm.at[pl.ds(s0 + c * _CHUNK, _CHUNK), :],
            posb[dc],
            psem[dc],
        )

    pos_load(0, 0).start()
    pos_load(1, 1).start()
    gather(0, 0, 0).start()
    gather(0, 1, 1).start()

    def pair_body(i, carry):
        for dc in range(2):
            c = 2 * i + dc
            for b in range(_B):
                p = b % 2
                gather(c, b, p).wait()
                if b == 0:
                    pos_load(c, dc).wait()
                if b < 2:
                    @pl.when(c > 0)
                    def _():
                        store(c, b, p).wait()
                else:
                    store(c, b, p).wait()

                src = rows[p]
                dst = outb[p]
                pv = posb[dc]

                def fma(j, acc):
                    off = j * _LANES
                    for r in range(_CHUNK):
                        dst[r, pl.ds(off, _LANES)] = (
                            src[r, pl.ds(off, _LANES)] * _SCALE
                            + pv[r, pl.ds(off, _LANES)]
                        )
                    return acc

                lax.fori_loop(0, _D // _LANES, fma, 0)

                if b < 2:
                    gather(c, b + 2, p).start()
                else:
                    cn = jnp.minimum(c + 1, _N_CHUNKS - 1)
                    gather(cn, b - 2, p).start()

                store(c, b, p).start()
                if b == _B - 1:
                    cn2 = jnp.minimum(c + 2, _N_CHUNKS - 1)
                    pos_load(cn2, dc).start()
        return carry

    lax.fori_loop(0, _N_CHUNKS // 2, pair_body, 0)

    gather(_N_CHUNKS - 1, 0, 0).wait()
    gather(_N_CHUNKS - 1, 1, 1).wait()
    store(_N_CHUNKS - 1, 2, 0).wait()
    store(_N_CHUNKS - 1, 3, 1).wait()
    pos_load(_N_CHUNKS - 1, 0).wait()
    pos_load(_N_CHUNKS - 1, 1).wait()


@jax.jit
def _embed(token_ids, token_table, pos_table):
    mesh = plsc.VectorSubcoreMesh(core_axis_name="c", subcore_axis_name="s")
    return pl.kernel(
        _body,
        out_type=jax.ShapeDtypeStruct((_B, _S, _D), jnp.float32),
        mesh=mesh,
        scratch_types=[
            pltpu.VMEM((_B, _S_PER_W), jnp.int32),
            pltpu.VMEM((_CHUNK, _D), jnp.float32),
            pltpu.VMEM((_CHUNK, _D), jnp.float32),
            pltpu.VMEM((_CHUNK, _D), jnp.float32),
            pltpu.VMEM((_CHUNK, _D), jnp.float32),
            pltpu.VMEM((_CHUNK, _D), jnp.float32),
            pltpu.VMEM((_CHUNK, _D), jnp.float32),
            pltpu.SemaphoreType.DMA,
            pltpu.SemaphoreType.DMA,
            pltpu.SemaphoreType.DMA,
            pltpu.SemaphoreType.DMA,
            pltpu.SemaphoreType.DMA,
            pltpu.SemaphoreType.DMA,
        ],
    )(token_ids, token_table, pos_table)


def kernel(token_ids, token_table, pos_table):
    return _embed(token_ids.astype(jnp.int32), token_table, pos_table)
